# baseline (speedup 1.0000x reference)
import functools

import jax
import jax.numpy as jnp
from jax import lax
from jax.experimental import pallas as pl
from jax.experimental.pallas import tpu as pltpu
from jax.experimental.pallas import tpu_sc as plsc

R, N = 128, 32768
K = 256
NC, NS, L = 2, 16, 16
NW = NC * NS
ROWS_PER_W = R // NW
CHUNKS = N // L
SEGS = 16
SEGCH = CHUNKS // SEGS
SEGCAP = SEGCH * L
HBITS = 11
HBUCKETS = 1 << HBITS
SHIFT = 32 - HBITS
T0 = 2.0
T0KEY = 0x40000000
RBITS = 19
RBUCKETS = 256


def _scalar(x):
    return x if x.ndim == 0 else x[0]


def _keys(v):
    b = lax.bitcast_convert_type(v, jnp.int32)
    return jnp.where(b >= 0, b, b ^ jnp.int32(0x7FFFFFFF))


def _body(x_hbm, out_hbm, rowa_v, rowb_v, hist_v, candv_v, outrow_v, sem):
    wid = lax.axis_index("s") * NC + lax.axis_index("c")
    iota16 = lax.iota(jnp.int32, L)
    ones = jnp.ones((L,), jnp.int32)
    zeros16 = jnp.zeros((L,), jnp.int32)

    def count_ge_ref(ref, tt, ncc, nvalid):
        def cnt(cc, acc):
            key = _keys(ref[pl.ds(cc * L, L)])
            valid = (cc * L + iota16) < nvalid
            ge = jnp.logical_and(key >= tt, valid)
            return acc + jnp.where(ge, ones, zeros16)

        return jnp.sum(lax.fori_loop(0, ncc, cnt, zeros16))

    def bit_search(ref, t0, nbits, ncc, nvalid, ktarget):
        def bit_body(i, t):
            tt = t | (jnp.int32(1) << (nbits - 1 - i))
            c_ge = count_ge_ref(ref, tt, ncc, nvalid)
            return jnp.where(c_ge >= ktarget, tt, t)

        return lax.fori_loop(0, nbits, bit_body, t0)

    def do_row(row, buf_v):
        def seg_body(i, offs):
            new = []
            for s in range(SEGS):
                c = s * SEGCH + i
                v = buf_v[pl.ds(c * L, L)]
                m = v >= jnp.float32(T0)
                plsc.store_compressed(
                    candv_v.at[pl.ds(s * SEGCAP + offs[s], L)], v,
                    mask=m)
                new.append(
                    offs[s] + _scalar(plsc.all_reduce_population_count(m)))
            return tuple(new)

        offs = lax.fori_loop(
            0, SEGCH, seg_body, (jnp.int32(0),) * SEGS)

        dst = jnp.int32(0)
        for s in range(SEGS):
            nc_s = offs[s]
            if s > 0:
                def mv(cc, _c, s=s, nc_s=nc_s, dst=dst):
                    chunk = candv_v[pl.ds(s * SEGCAP + cc * L, L)]
                    valid = (cc * L + iota16) < nc_s
                    plsc.store_compressed(
                        candv_v.at[pl.ds(dst + cc * L, L)], chunk,
                        mask=valid)
                    return 0

                lax.fori_loop(0, (nc_s + L - 1) // L, mv, 0)
            dst = dst + nc_s
        nc0 = dst

        def path_fast(_):
            ncc = (nc0 + (L - 1)) // L

            def zero_hist(i, _c):
                hist_v[pl.ds(i * L, L)] = zeros16
                return 0

            lax.fori_loop(0, RBUCKETS // L, zero_hist, 0)

            def hist_pass(cc, _c):
                key = _keys(candv_v[pl.ds(cc * L, L)])
                valid = (cc * L + iota16) < nc0
                bkt = jnp.minimum(
                    (key - jnp.int32(T0KEY)) >> RBITS,
                    jnp.int32(RBUCKETS - 1))
                plsc.addupdate_scatter(
                    hist_v, [bkt], jnp.where(valid, ones, zeros16))
                return 0

            lax.fori_loop(0, ncc, hist_pass, 0)

            def scan_body(i, st):
                acc, b8 = st
                cb = (RBUCKETS // L - 1) - i
                h = hist_v[pl.ds(cb * L, L)]
                hr = lax.rev(h, (0,))
                cumr = plsc.cumsum(hr)
                tot = cumr[15]
                cross = (acc + cumr) >= K
                take = jnp.logical_and(acc + tot >= K, b8 < 0)
                f = _scalar(plsc.all_reduce_ffs(cross))
                b8 = jnp.where(take, cb * L + (15 - f), b8)
                return (acc + tot, b8)

            _, b8 = lax.fori_loop(
                0, RBUCKETS // L, scan_body,
                (jnp.int32(0), jnp.int32(-1)))

            def refine_sub(_):
                def sub_pass(cc, st):
                    off2, nabove = st
                    v = candv_v[pl.ds(cc * L, L)]
                    key = _keys(v)
                    valid = (cc * L + iota16) < nc0
                    bkt = jnp.minimum(
                        (key - jnp.int32(T0KEY)) >> RBITS,
                        jnp.int32(RBUCKETS - 1))
                    m = jnp.logical_and(bkt == b8, valid)
                    gt = jnp.logical_and(bkt > b8, valid)
                    plsc.store_compressed(
                        buf_v.at[pl.ds(off2, L)], v, mask=m)
                    return (
                        off2 + _scalar(
                            plsc.all_reduce_population_count(m)),
                        nabove + _scalar(
                            plsc.all_reduce_population_count(gt)))

                nsub, n_above = lax.fori_loop(
                    0, ncc, sub_pass, (jnp.int32(0), jnp.int32(0)))
                nsc = (nsub + (L - 1)) // L
                t1 = jnp.int32(T0KEY) + (b8 << RBITS)
                ktarget = K - n_above
                tkey = bit_search(buf_v, t1, RBITS, nsc, nsub, ktarget)
                c_gt = n_above + count_ge_ref(
                    buf_v, tkey + 1, nsc, nsub)
                c_ge = n_above + count_ge_ref(buf_v, tkey, nsc, nsub)
                return tkey, c_gt, c_ge

            def refine_full(_):
                tkey = bit_search(
                    candv_v, jnp.int32(T0KEY), 30, ncc, nc0, K)
                return (tkey,
                        count_ge_ref(candv_v, tkey + 1, ncc, nc0),
                        count_ge_ref(candv_v, tkey, ncc, nc0))

            tkey, c_gt, c_ge = lax.cond(
                b8 < jnp.int32(RBUCKETS - 1), refine_sub, refine_full, 0)
            return tkey, nc0, c_gt, c_ge

        def path_exact(_):
            def zero_hist(i, _c):
                hist_v[pl.ds(i * L, L)] = zeros16
                return 0

            lax.fori_loop(0, HBUCKETS // L, zero_hist, 0)

            def hist_pass(c, _c):
                key = _keys(buf_v[pl.ds(c * L, L)])
                bkt = (key >> SHIFT) + (HBUCKETS // 2)
                plsc.addupdate_scatter(hist_v, [bkt], ones)
                return 0

            lax.fori_loop(0, CHUNKS, hist_pass, 0)

            def scan_body(i, st):
                acc, bstar = st
                cb = (HBUCKETS // L - 1) - i
                h = hist_v[pl.ds(cb * L, L)]
                hr = lax.rev(h, (0,))
                cumr = plsc.cumsum(hr)
                tot = cumr[15]
                cross = (acc + cumr) >= K
                take = jnp.logical_and(acc + tot >= K, bstar < 0)
                f = _scalar(plsc.all_reduce_ffs(cross))
                bstar = jnp.where(take, cb * L + (15 - f), bstar)
                return (acc + tot, bstar)

            _, bstar = lax.fori_loop(
                0, HBUCKETS // L, scan_body,
                (jnp.int32(0), jnp.int32(-1)))

            def cand_pass(c, off):
                v = buf_v[pl.ds(c * L, L)]
                key = _keys(v)
                m = ((key >> SHIFT) + (HBUCKETS // 2)) >= bstar
                plsc.store_compressed(
                    candv_v.at[pl.ds(off, L)], v, mask=m)
                return off + _scalar(plsc.all_reduce_population_count(m))

            nc = lax.fori_loop(0, CHUNKS, cand_pass, jnp.int32(0))
            ncc = (nc + (L - 1)) // L
            t0 = (bstar - HBUCKETS // 2) << SHIFT
            tkey = bit_search(candv_v, t0, SHIFT, ncc, nc, K)
            return (tkey, nc,
                    count_ge_ref(candv_v, tkey + 1, ncc, nc),
                    count_ge_ref(candv_v, tkey, ncc, nc))

        tkey, nc, c_gt, c_ge = lax.cond(
            nc0 >= K, path_fast, path_exact, 0)
        ncc = (nc + (L - 1)) // L
        needed_eq = K - c_gt

        def simple_sel(_):
            def sp(cc, off):
                v = candv_v[pl.ds(cc * L, L)]
                key = _keys(v)
                valid = (cc * L + iota16) < nc
                m = jnp.logical_and(key >= tkey, valid)
                plsc.store_compressed(
                    outrow_v.at[pl.ds(off, L)], v, mask=m)
                return off + _scalar(plsc.all_reduce_population_count(m))

            lax.fori_loop(0, ncc, sp, jnp.int32(0))
            return 0

        def tie_sel(_):
            def sp(cc, st):
                off, cnt_eq = st
                v = candv_v[pl.ds(cc * L, L)]
                key = _keys(v)
                valid = (cc * L + iota16) < nc
                meq = jnp.logical_and(key == tkey, valid)
                cum = plsc.cumsum(meq.astype(jnp.int32))
                sel_eq = jnp.logical_and(
                    meq, (cnt_eq + cum) <= needed_eq)
                m = jnp.logical_or(
                    jnp.logical_and(key > tkey, valid), sel_eq)
                plsc.store_compressed(
                    outrow_v.at[pl.ds(off, L)], v, mask=m)
                return (off + _scalar(
                            plsc.all_reduce_population_count(m)),
                        cnt_eq + cum[15])

            lax.fori_loop(0, ncc, sp, (jnp.int32(0), jnp.int32(0)))
            return 0

        lax.cond(c_ge == K, simple_sel, tie_sel, 0)
        pltpu.sync_copy(outrow_v.at[pl.ds(0, K)], out_hbm.at[row])

    row0 = wid * ROWS_PER_W
    pltpu.sync_copy(x_hbm.at[row0], rowa_v)
    bufs = [rowa_v, rowb_v]
    handle = None
    for j in range(ROWS_PER_W):
        if j + 1 < ROWS_PER_W:
            handle = pltpu.async_copy(
                x_hbm.at[row0 + j + 1], bufs[(j + 1) % 2], sem)
        do_row(row0 + j, bufs[j % 2])
        if handle is not None:
            handle.wait()
            handle = None


_mesh = plsc.VectorSubcoreMesh(
    core_axis_name="c", subcore_axis_name="s", num_cores=NC, num_subcores=NS)

_kmax = pl.kernel(
    _body,
    out_type=jax.ShapeDtypeStruct((R, K), jnp.float32),
    mesh=_mesh,
    scratch_types=[
        pltpu.VMEM((N,), jnp.float32),
        pltpu.VMEM((N,), jnp.float32),
        pltpu.VMEM((HBUCKETS,), jnp.int32),
        pltpu.VMEM((N,), jnp.float32),
        pltpu.VMEM((K + L,), jnp.float32),
        pltpu.SemaphoreType.DMA,
    ],
    compiler_params=pltpu.CompilerParams(needs_layout_passes=False),
)


@jax.jit
def kernel(x):
    return _kmax(x)


# speedup vs baseline: 1.3568x; 1.3568x over previous
import functools

import jax
import jax.numpy as jnp
from jax import lax
from jax.experimental import pallas as pl
from jax.experimental.pallas import tpu as pltpu
from jax.experimental.pallas import tpu_sc as plsc

R, N = 128, 32768
K = 256
NC, NS, L = 2, 16, 16
NW = NC * NS
ROWS_PER_W = R // NW
CHUNKS = N // L
SEGS = 16
SEGCH = CHUNKS // SEGS
SEGW = SEGCH * L
SEGCAP = 1024
T0 = 2.0
T0KEY = 0x40000000
RBITS = 19
RBUCKETS = 256
IMIN = -2147483648


def _scalar(x):
    return x if x.ndim == 0 else x[0]


def _keys(v):
    b = lax.bitcast_convert_type(v, jnp.int32)
    return jnp.where(b >= 0, b, b ^ jnp.int32(0x7FFFFFFF))


def _body(x_hbm, out_hbm, rowa_v, rowb_v, hist_v, candv_v, outrow_v, sem,
          *segs_v):
    wid = lax.axis_index("s") * NC + lax.axis_index("c")
    iota16 = lax.iota(jnp.int32, L)
    ones = jnp.ones((L,), jnp.int32)
    zeros16 = jnp.zeros((L,), jnp.int32)

    def count_ge_ref(ref, tt, ncc, nvalid):
        def cnt(cc, acc):
            key = _keys(ref[pl.ds(cc * L, L)])
            valid = (cc * L + iota16) < nvalid
            ge = jnp.logical_and(key >= tt, valid)
            return acc + jnp.where(ge, ones, zeros16)

        return jnp.sum(lax.fori_loop(0, ncc, cnt, zeros16))

    def bit_search(ref, t0, nbits, ncc, nvalid, ktarget):
        def bit_body(i, t):
            tt = t | (jnp.int32(1) << (nbits - 1 - i))
            c_ge = count_ge_ref(ref, tt, ncc, nvalid)
            return jnp.where(c_ge >= ktarget, tt, t)

        return lax.fori_loop(0, nbits, bit_body, t0)

    def emit_selection(ref, nvalid, ncc, tkey, c_gt, c_ge):
        def simple_sel(_):
            def sp(cc, off):
                v = ref[pl.ds(cc * L, L)]
                key = _keys(v)
                valid = (cc * L + iota16) < nvalid
                m = jnp.logical_and(key >= tkey, valid)
                plsc.store_compressed(
                    outrow_v.at[pl.ds(off, L)], v, mask=m)
                return off + _scalar(plsc.all_reduce_population_count(m))

            lax.fori_loop(0, ncc, sp, jnp.int32(0))
            return 0

        def tie_sel(_):
            needed_eq = K - c_gt

            def sp(cc, st):
                off, cnt_eq = st
                v = ref[pl.ds(cc * L, L)]
                key = _keys(v)
                valid = (cc * L + iota16) < nvalid
                meq = jnp.logical_and(key == tkey, valid)
                cum = plsc.cumsum(meq.astype(jnp.int32))
                sel_eq = jnp.logical_and(
                    meq, (cnt_eq + cum) <= needed_eq)
                m = jnp.logical_or(
                    jnp.logical_and(key > tkey, valid), sel_eq)
                plsc.store_compressed(
                    outrow_v.at[pl.ds(off, L)], v, mask=m)
                return (off + _scalar(
                            plsc.all_reduce_population_count(m)),
                        cnt_eq + cum[15])

            lax.fori_loop(0, ncc, sp, (jnp.int32(0), jnp.int32(0)))
            return 0

        lax.cond(c_ge == K, simple_sel, tie_sel, 0)

    def do_row(row, buf_v):
        @plsc.parallel_loop(0, SEGCH, step=1, unroll=1,
                            carry=(jnp.int32(0),) * SEGS)
        def offs(i, offs):
            new = []
            for s in range(SEGS):
                c = s * SEGCH + i
                v = buf_v[pl.ds(c * L, L)]
                m = v >= jnp.float32(T0)
                off_c = jnp.minimum(offs[s], jnp.int32(SEGCAP - L))
                plsc.store_compressed(
                    segs_v[s].at[pl.ds(off_c, L)], v, mask=m)
                new.append(
                    offs[s] + _scalar(plsc.all_reduce_population_count(m)))
            return tuple(new)

        ovf = offs[0] > (SEGCAP - L)
        for s in range(1, SEGS):
            ovf = jnp.logical_or(ovf, offs[s] > (SEGCAP - L))

        dst = jnp.int32(0)
        for s in range(SEGS):
            nc_s = jnp.minimum(offs[s], jnp.int32(SEGCAP))

            def mv(cc, _c, s=s, nc_s=nc_s, dst=dst):
                chunk = segs_v[s][pl.ds(cc * L, L)]
                valid = (cc * L + iota16) < nc_s
                plsc.store_compressed(
                    candv_v.at[pl.ds(dst + cc * L, L)], chunk,
                    mask=valid)
                return 0

            lax.fori_loop(0, (nc_s + L - 1) // L, mv, 0)
            dst = dst + nc_s
        nc0 = dst

        def path_fast(_):
            ncc = (nc0 + (L - 1)) // L

            def zero_hist(i, _c):
                hist_v[pl.ds(i * L, L)] = zeros16
                return 0

            lax.fori_loop(0, RBUCKETS // L, zero_hist, 0)

            def hist_pass(cc, _c):
                key = _keys(candv_v[pl.ds(cc * L, L)])
                valid = (cc * L + iota16) < nc0
                bkt = jnp.minimum(
                    (key - jnp.int32(T0KEY)) >> RBITS,
                    jnp.int32(RBUCKETS - 1))
                plsc.addupdate_scatter(
                    hist_v, [bkt], jnp.where(valid, ones, zeros16))
                return 0

            lax.fori_loop(0, ncc, hist_pass, 0)

            def scan_body(i, st):
                acc, b8 = st
                cb = (RBUCKETS // L - 1) - i
                h = hist_v[pl.ds(cb * L, L)]
                hr = lax.rev(h, (0,))
                cumr = plsc.cumsum(hr)
                tot = cumr[15]
                cross = (acc + cumr) >= K
                take = jnp.logical_and(acc + tot >= K, b8 < 0)
                f = _scalar(plsc.all_reduce_ffs(cross))
                b8 = jnp.where(take, cb * L + (15 - f), b8)
                return (acc + tot, b8)

            _, b8 = lax.fori_loop(
                0, RBUCKETS // L, scan_body,
                (jnp.int32(0), jnp.int32(-1)))

            def refine_sub(_):
                def sub_pass(cc, st):
                    off2, nabove = st
                    v = candv_v[pl.ds(cc * L, L)]
                    key = _keys(v)
                    valid = (cc * L + iota16) < nc0
                    bkt = jnp.minimum(
                        (key - jnp.int32(T0KEY)) >> RBITS,
                        jnp.int32(RBUCKETS - 1))
                    m = jnp.logical_and(bkt == b8, valid)
                    gt = jnp.logical_and(bkt > b8, valid)
                    plsc.store_compressed(
                        buf_v.at[pl.ds(off2, L)], v, mask=m)
                    return (
                        off2 + _scalar(
                            plsc.all_reduce_population_count(m)),
                        nabove + _scalar(
                            plsc.all_reduce_population_count(gt)))

                nsub, n_above = lax.fori_loop(
                    0, ncc, sub_pass, (jnp.int32(0), jnp.int32(0)))
                nsc = (nsub + (L - 1)) // L
                t1 = jnp.int32(T0KEY) + (b8 << RBITS)
                ktarget = K - n_above
                tkey = bit_search(buf_v, t1, RBITS, nsc, nsub, ktarget)
                c_gt = n_above + count_ge_ref(
                    buf_v, tkey + 1, nsc, nsub)
                c_ge = n_above + count_ge_ref(buf_v, tkey, nsc, nsub)
                return tkey, c_gt, c_ge

            def refine_full(_):
                tkey = bit_search(
                    candv_v, jnp.int32(T0KEY), 30, ncc, nc0, K)
                return (tkey,
                        count_ge_ref(candv_v, tkey + 1, ncc, nc0),
                        count_ge_ref(candv_v, tkey, ncc, nc0))

            tkey, c_gt, c_ge = lax.cond(
                b8 < jnp.int32(RBUCKETS - 1), refine_sub, refine_full, 0)
            emit_selection(candv_v, nc0, ncc, tkey, c_gt, c_ge)
            return 0

        def path_exact(_):
            c_pos = count_ge_ref(buf_v, jnp.int32(0), CHUNKS, N)

            def pos_case(_c):
                return bit_search(buf_v, jnp.int32(0), 31, CHUNKS, N, K)

            def neg_case(_c):
                return bit_search(buf_v, jnp.int32(IMIN), 31, CHUNKS, N, K)

            tkey = lax.cond(c_pos >= K, pos_case, neg_case, 0)
            c_gt = count_ge_ref(buf_v, tkey + 1, CHUNKS, N)
            c_ge = count_ge_ref(buf_v, tkey, CHUNKS, N)
            emit_selection(buf_v, N, CHUNKS, tkey, c_gt, c_ge)
            return 0

        use_fast = jnp.logical_and(nc0 >= K, jnp.logical_not(ovf))
        lax.cond(use_fast, path_fast, path_exact, 0)
        pltpu.sync_copy(outrow_v.at[pl.ds(0, K)], out_hbm.at[row])

    row0 = wid * ROWS_PER_W
    pltpu.sync_copy(x_hbm.at[row0], rowa_v)
    bufs = [rowa_v, rowb_v]
    handle = None
    for j in range(ROWS_PER_W):
        if j + 1 < ROWS_PER_W:
            handle = pltpu.async_copy(
                x_hbm.at[row0 + j + 1], bufs[(j + 1) % 2], sem)
        do_row(row0 + j, bufs[j % 2])
        if handle is not None:
            handle.wait()
            handle = None


_mesh = plsc.VectorSubcoreMesh(
    core_axis_name="c", subcore_axis_name="s", num_cores=NC, num_subcores=NS)

_kmax = pl.kernel(
    _body,
    out_type=jax.ShapeDtypeStruct((R, K), jnp.float32),
    mesh=_mesh,
    scratch_types=[
        pltpu.VMEM((N,), jnp.float32),
        pltpu.VMEM((N,), jnp.float32),
        pltpu.VMEM((RBUCKETS,), jnp.int32),
        pltpu.VMEM((SEGS * SEGCAP,), jnp.float32),
        pltpu.VMEM((K + L,), jnp.float32),
        pltpu.SemaphoreType.DMA,
    ] + [pltpu.VMEM((SEGCAP,), jnp.float32) for _ in range(SEGS)],
    compiler_params=pltpu.CompilerParams(needs_layout_passes=False),
)


@jax.jit
def kernel(x):
    return _kmax(x)


# speedup vs baseline: 2.2188x; 1.6353x over previous
import functools

import jax
import jax.numpy as jnp
from jax import lax
from jax.experimental import pallas as pl
from jax.experimental.pallas import tpu as pltpu
from jax.experimental.pallas import tpu_sc as plsc

R, N = 128, 32768
K = 256
NC, NS, L = 2, 16, 16
NW = NC * NS
ROWS_PER_W = R // NW
CHUNKS = N // L
T0 = 2.0
T0KEY = 0x40000000
RBITS = 19
RBUCKETS = 256
IMIN = -2147483648


def _scalar(x):
    return x if x.ndim == 0 else x[0]


def _keys(v):
    b = lax.bitcast_convert_type(v, jnp.int32)
    return jnp.where(b >= 0, b, b ^ jnp.int32(0x7FFFFFFF))


def _body(x_hbm, out_hbm, rowa_v, rowb_v, hist_v, candv_v, outrow_v, sem):
    wid = lax.axis_index("s") * NC + lax.axis_index("c")
    iota16 = lax.iota(jnp.int32, L)
    ones = jnp.ones((L,), jnp.int32)
    zeros16 = jnp.zeros((L,), jnp.int32)

    def count_ge_ref(ref, tt, ncc, nvalid):
        def cnt(cc, acc):
            key = _keys(ref[pl.ds(cc * L, L)])
            valid = (cc * L + iota16) < nvalid
            ge = jnp.logical_and(key >= tt, valid)
            return acc + jnp.where(ge, ones, zeros16)

        return jnp.sum(lax.fori_loop(0, ncc, cnt, zeros16))

    def bit_search(ref, t0, nbits, ncc, nvalid, ktarget):
        def bit_body(i, t):
            tt = t | (jnp.int32(1) << (nbits - 1 - i))
            c_ge = count_ge_ref(ref, tt, ncc, nvalid)
            return jnp.where(c_ge >= ktarget, tt, t)

        return lax.fori_loop(0, nbits, bit_body, t0)

    def emit_selection(ref, nvalid, ncc, tkey, c_gt, c_ge):
        def simple_sel(_):
            def sp(cc, off):
                v = ref[pl.ds(cc * L, L)]
                key = _keys(v)
                valid = (cc * L + iota16) < nvalid
                m = jnp.logical_and(key >= tkey, valid)
                plsc.store_compressed(
                    outrow_v.at[pl.ds(off, L)], v, mask=m)
                return off + _scalar(plsc.all_reduce_population_count(m))

            lax.fori_loop(0, ncc, sp, jnp.int32(0))
            return 0

        def tie_sel(_):
            needed_eq = K - c_gt

            def sp(cc, st):
                off, cnt_eq = st
                v = ref[pl.ds(cc * L, L)]
                key = _keys(v)
                valid = (cc * L + iota16) < nvalid
                meq = jnp.logical_and(key == tkey, valid)
                cum = plsc.cumsum(meq.astype(jnp.int32))
                sel_eq = jnp.logical_and(
                    meq, (cnt_eq + cum) <= needed_eq)
                m = jnp.logical_or(
                    jnp.logical_and(key > tkey, valid), sel_eq)
                plsc.store_compressed(
                    outrow_v.at[pl.ds(off, L)], v, mask=m)
                return (off + _scalar(
                            plsc.all_reduce_population_count(m)),
                        cnt_eq + cum[15])

            lax.fori_loop(0, ncc, sp, (jnp.int32(0), jnp.int32(0)))
            return 0

        lax.cond(c_ge == K, simple_sel, tie_sel, 0)

    def do_row(row, buf_v):
        @plsc.parallel_loop(0, CHUNKS, step=1, unroll=8,
                            carry=jnp.int32(0))
        def nc0(c, off):
            v = buf_v[pl.ds(c * L, L)]
            m = v >= jnp.float32(T0)
            plsc.store_compressed(candv_v.at[pl.ds(off, L)], v, mask=m)
            return off + _scalar(plsc.all_reduce_population_count(m))

        def path_fast(_):
            ncc = (nc0 + (L - 1)) // L

            def zero_hist(i, _c):
                hist_v[pl.ds(i * L, L)] = zeros16
                return 0

            lax.fori_loop(0, RBUCKETS // L, zero_hist, 0)

            def hist_pass(cc, _c):
                key = _keys(candv_v[pl.ds(cc * L, L)])
                valid = (cc * L + iota16) < nc0
                bkt = jnp.minimum(
                    (key - jnp.int32(T0KEY)) >> RBITS,
                    jnp.int32(RBUCKETS - 1))
                plsc.addupdate_scatter(
                    hist_v, [bkt], jnp.where(valid, ones, zeros16))
                return 0

            lax.fori_loop(0, ncc, hist_pass, 0)

            def scan_body(i, st):
                acc, b8 = st
                cb = (RBUCKETS // L - 1) - i
                h = hist_v[pl.ds(cb * L, L)]
                hr = lax.rev(h, (0,))
                cumr = plsc.cumsum(hr)
                tot = cumr[15]
                cross = (acc + cumr) >= K
                take = jnp.logical_and(acc + tot >= K, b8 < 0)
                f = _scalar(plsc.all_reduce_ffs(cross))
                b8 = jnp.where(take, cb * L + (15 - f), b8)
                return (acc + tot, b8)

            _, b8 = lax.fori_loop(
                0, RBUCKETS // L, scan_body,
                (jnp.int32(0), jnp.int32(-1)))

            def refine_sub(_):
                def sub_pass(cc, st):
                    off2, nabove = st
                    v = candv_v[pl.ds(cc * L, L)]
                    key = _keys(v)
                    valid = (cc * L + iota16) < nc0
                    bkt = jnp.minimum(
                        (key - jnp.int32(T0KEY)) >> RBITS,
                        jnp.int32(RBUCKETS - 1))
                    m = jnp.logical_and(bkt == b8, valid)
                    gt = jnp.logical_and(bkt > b8, valid)
                    plsc.store_compressed(
                        buf_v.at[pl.ds(off2, L)], v, mask=m)
                    return (
                        off2 + _scalar(
                            plsc.all_reduce_population_count(m)),
                        nabove + _scalar(
                            plsc.all_reduce_population_count(gt)))

                nsub, n_above = lax.fori_loop(
                    0, ncc, sub_pass, (jnp.int32(0), jnp.int32(0)))
                nsc = (nsub + (L - 1)) // L
                t1 = jnp.int32(T0KEY) + (b8 << RBITS)
                ktarget = K - n_above
                tkey = bit_search(buf_v, t1, RBITS, nsc, nsub, ktarget)
                c_gt = n_above + count_ge_ref(
                    buf_v, tkey + 1, nsc, nsub)
                c_ge = n_above + count_ge_ref(buf_v, tkey, nsc, nsub)
                return tkey, c_gt, c_ge

            def refine_full(_):
                tkey = bit_search(
                    candv_v, jnp.int32(T0KEY), 30, ncc, nc0, K)
                return (tkey,
                        count_ge_ref(candv_v, tkey + 1, ncc, nc0),
                        count_ge_ref(candv_v, tkey, ncc, nc0))

            tkey, c_gt, c_ge = lax.cond(
                b8 < jnp.int32(RBUCKETS - 1), refine_sub, refine_full, 0)
            emit_selection(candv_v, nc0, ncc, tkey, c_gt, c_ge)
            return 0

        def path_exact(_):
            c_pos = count_ge_ref(buf_v, jnp.int32(0), CHUNKS, N)

            def pos_case(_c):
                return bit_search(buf_v, jnp.int32(0), 31, CHUNKS, N, K)

            def neg_case(_c):
                return bit_search(buf_v, jnp.int32(IMIN), 31, CHUNKS, N, K)

            tkey = lax.cond(c_pos >= K, pos_case, neg_case, 0)
            c_gt = count_ge_ref(buf_v, tkey + 1, CHUNKS, N)
            c_ge = count_ge_ref(buf_v, tkey, CHUNKS, N)
            emit_selection(buf_v, N, CHUNKS, tkey, c_gt, c_ge)
            return 0

        lax.cond(nc0 >= K, path_fast, path_exact, 0)
        pltpu.sync_copy(outrow_v.at[pl.ds(0, K)], out_hbm.at[row])

    row0 = wid * ROWS_PER_W
    pltpu.sync_copy(x_hbm.at[row0], rowa_v)
    bufs = [rowa_v, rowb_v]
    handle = None
    for j in range(ROWS_PER_W):
        if j + 1 < ROWS_PER_W:
            handle = pltpu.async_copy(
                x_hbm.at[row0 + j + 1], bufs[(j + 1) % 2], sem)
        do_row(row0 + j, bufs[j % 2])
        if handle is not None:
            handle.wait()
            handle = None


_mesh = plsc.VectorSubcoreMesh(
    core_axis_name="c", subcore_axis_name="s", num_cores=NC, num_subcores=NS)

_kmax = pl.kernel(
    _body,
    out_type=jax.ShapeDtypeStruct((R, K), jnp.float32),
    mesh=_mesh,
    scratch_types=[
        pltpu.VMEM((N,), jnp.float32),
        pltpu.VMEM((N,), jnp.float32),
        pltpu.VMEM((RBUCKETS,), jnp.int32),
        pltpu.VMEM((N,), jnp.float32),
        pltpu.VMEM((K + L,), jnp.float32),
        pltpu.SemaphoreType.DMA,
    ],
    compiler_params=pltpu.CompilerParams(needs_layout_passes=False),
)


@jax.jit
def kernel(x):
    return _kmax(x)
